# baseline (speedup 1.0000x reference)
import functools

import jax
import jax.numpy as jnp
from jax import lax
from jax.experimental import pallas as pl
from jax.experimental.pallas import tpu as pltpu
from jax.experimental.pallas import tpu_sc as plsc

_LANES = 16


def _sc_gather_body(table_hbm, idx_hbm, out_hbm, table_v, idx_v, out_v):
    head = lax.axis_index("s")
    half = lax.axis_index("c")
    chunk = idx_v.shape[0]
    base = half * chunk
    pltpu.sync_copy(table_hbm.at[head], table_v)
    pltpu.sync_copy(idx_hbm.at[pl.ds(base, chunk)], idx_v)

    def body(i, carry):
        sl = pl.ds(i * _LANES, _LANES)
        out_v[sl] = plsc.load_gather(table_v, [idx_v[sl]])
        return carry

    lax.fori_loop(0, chunk // _LANES, body, 0)
    pltpu.sync_copy(out_v, out_hbm.at[head, pl.ds(base, chunk)])


def _sc_gather(table_pad, idx_flat_pad, num_heads, p_pad):
    chunk = p_pad // 2
    mesh = plsc.VectorSubcoreMesh(core_axis_name="c", subcore_axis_name="s")
    return pl.kernel(
        _sc_gather_body,
        out_type=jax.ShapeDtypeStruct((num_heads, p_pad), jnp.float32),
        mesh=mesh,
        compiler_params=pltpu.CompilerParams(
            use_tc_tiling_on_sc=False, needs_layout_passes=False
        ),
        scratch_types=[
            pltpu.VMEM((table_pad.shape[1],), jnp.float32),
            pltpu.VMEM((chunk,), jnp.int32),
            pltpu.VMEM((chunk,), jnp.float32),
        ],
    )(table_pad, idx_flat_pad)


def _add_body(x_ref, pos_ref, o_ref):
    o_ref[...] = x_ref[...] + pos_ref[...]


def kernel(inputs, relative_position_bias_table, relative_position_index):
    b, h, s_q, s_k = inputs.shape
    p = s_q * s_k
    p_pad = ((p + 31) // 32) * 32

    idx = relative_position_index[:s_q, :s_k].astype(jnp.int32).reshape(-1)
    idx_pad = jnp.pad(idx, (0, p_pad - p))
    nrd = relative_position_bias_table.shape[1]
    nrd_pad = ((nrd + 7) // 8) * 8
    table_pad = jnp.pad(relative_position_bias_table, ((0, 0), (0, nrd_pad - nrd)))

    pos_pad = _sc_gather(table_pad, idx_pad, h, p_pad)
    pos = pos_pad[:, :p]

    x = inputs.reshape(b, h, p)
    out = pl.pallas_call(
        _add_body,
        out_shape=jax.ShapeDtypeStruct((b, h, p), jnp.float32),
        grid=(b,),
        in_specs=[
            pl.BlockSpec((1, h, p), lambda i: (i, 0, 0)),
            pl.BlockSpec((h, p), lambda i: (0, 0)),
        ],
        out_specs=pl.BlockSpec((1, h, p), lambda i: (i, 0, 0)),
    )(x, pos)
    return out.reshape(b, h, s_q, s_k)


# speedup vs baseline: 1.0032x; 1.0032x over previous
import jax
import jax.numpy as jnp
from jax import lax
from jax.experimental import pallas as pl
from jax.experimental.pallas import tpu as pltpu
from jax.experimental.pallas import tpu_sc as plsc

_LANES = 16


def _sc_gather_body(table_hbm, idx_hbm, out_hbm, table_v, idx_v, out_v):
    head = lax.axis_index("s")
    half = lax.axis_index("c")
    row = table_v.shape[0]
    chunk = idx_v.shape[0]
    p_pad = 2 * chunk
    pltpu.sync_copy(idx_hbm.at[pl.ds(half * chunk, chunk)], idx_v)
    pltpu.sync_copy(table_hbm.at[pl.ds(head * row, row)], table_v)

    def body(i, carry):
        sl = pl.ds(i * _LANES, _LANES)
        out_v[sl] = plsc.load_gather(table_v, [idx_v[sl]])
        return carry

    lax.fori_loop(0, chunk // _LANES, body, 0)
    pltpu.sync_copy(out_v, out_hbm.at[pl.ds(head * p_pad + half * chunk, chunk)])


def _sc_gather(table_flat, idx_flat_pad, num_heads, row, p_pad):
    chunk = p_pad // 2
    mesh = plsc.VectorSubcoreMesh(core_axis_name="c", subcore_axis_name="s")
    return pl.kernel(
        _sc_gather_body,
        out_type=jax.ShapeDtypeStruct((num_heads * p_pad,), jnp.float32),
        mesh=mesh,
        compiler_params=pltpu.CompilerParams(needs_layout_passes=False),
        scratch_types=[
            pltpu.VMEM((row,), jnp.float32),
            pltpu.VMEM((chunk,), jnp.int32),
            pltpu.VMEM((chunk,), jnp.float32),
        ],
    )(table_flat, idx_flat_pad)


def _add_body(x_ref, pos_ref, o_ref):
    o_ref[...] = x_ref[...] + pos_ref[...]


def kernel(inputs, relative_position_bias_table, relative_position_index):
    b, h, s_q, s_k = inputs.shape
    p = s_q * s_k
    p_pad = ((p + 255) // 256) * 256

    idx = relative_position_index[:s_q, :s_k].astype(jnp.int32).reshape(-1)
    idx_pad = jnp.pad(idx, (0, p_pad - p))
    nrd = relative_position_bias_table.shape[1]
    row = ((nrd + 127) // 128) * 128
    table_flat = jnp.pad(
        relative_position_bias_table, ((0, 0), (0, row - nrd))
    ).reshape(-1)

    pos_flat = _sc_gather(table_flat, idx_pad, h, row, p_pad)
    pos = pos_flat.reshape(h, p_pad)[:, :p]

    x = inputs.reshape(b, h, p)
    out = pl.pallas_call(
        _add_body,
        out_shape=jax.ShapeDtypeStruct((b, h, p), jnp.float32),
        grid=(b,),
        in_specs=[
            pl.BlockSpec((1, h, p), lambda i: (i, 0, 0)),
            pl.BlockSpec((h, p), lambda i: (0, 0)),
        ],
        out_specs=pl.BlockSpec((1, h, p), lambda i: (i, 0, 0)),
    )(x, pos)
    return out.reshape(b, h, s_q, s_k)
